# baseline (speedup 1.0000x reference)
import functools

import jax
import jax.numpy as jnp
from jax import lax
from jax.experimental import pallas as pl
from jax.experimental.pallas import tpu as pltpu
from jax.experimental.pallas import tpu_sc as plsc

B = 256
T = 192
D = 512
H = 512
E = 1024
K = 58
KP = 64
ROWS = B * KP
NW = 32
NSEL = B * K


def _excl_cumsum_lanes(x):
    inc = x
    k = 1
    while k < T:
        sh = jnp.concatenate(
            [jnp.zeros((B, k), inc.dtype), inc[:, : T - k]], axis=1
        )
        inc = inc + sh
        k *= 2
    return inc - x


def _select_body(att_ref, pos_ref):
    a = att_ref[...]

    def bis(_, carry):
        lo, hi = carry
        mid = (lo + hi) >> 1
        tau = lax.bitcast_convert_type(mid, jnp.float32)
        cnt = jnp.sum(jnp.where(a > tau, 1.0, 0.0), axis=1, keepdims=True)
        pred = cnt < float(K)
        return jnp.where(pred, lo, mid + 1), jnp.where(pred, mid, hi)

    lo0 = jnp.zeros((B, 1), jnp.int32)
    hi0 = jnp.full((B, 1), 0x3F800000, jnp.int32)
    lo, hi = lax.fori_loop(0, 31, bis, (lo0, hi0))
    vstar = lax.bitcast_convert_type(lo, jnp.float32)
    gt = a > vstar
    eq = a == vstar
    gtn = jnp.where(gt, 1, 0)
    cnt_gt = jnp.sum(gtn, axis=1, keepdims=True)
    eqn = jnp.where(eq, 1, 0)
    eqpre = _excl_cumsum_lanes(eqn)
    sel = gt | (eq & (eqpre < (K - cnt_gt)))
    seln = jnp.where(sel, 1, 0)
    pos = _excl_cumsum_lanes(seln)
    iota_t = lax.broadcasted_iota(jnp.int32, (B, T), 1)
    gidx = iota_t + T * lax.broadcasted_iota(jnp.int32, (B, T), 0)
    vals = jnp.where(sel, gidx, 0)
    disp = jnp.where(sel, iota_t - pos, 0)
    bit = 1
    while bit < T:
        sv = jnp.concatenate([vals[:, bit:], jnp.zeros((B, bit), jnp.int32)], axis=1)
        sd = jnp.concatenate([disp[:, bit:], jnp.zeros((B, bit), jnp.int32)], axis=1)
        take = (sd & bit) != 0
        moved = (disp & bit) != 0
        vals = jnp.where(take, sv, jnp.where(moved, 0, vals))
        disp = jnp.where(take, sd - bit, jnp.where(moved, 0, disp))
        bit *= 2
    pad = T * lax.broadcasted_iota(jnp.int32, (B, KP - K), 0)
    pos_ref[...] = jnp.concatenate([vals[:, :K], pad], axis=1)


_select = pl.pallas_call(
    _select_body,
    out_shape=jax.ShapeDtypeStruct((B, KP), jnp.int32),
)


SPW = B // NW


def _gather_body(table_hbm, idx_hbm, out_hbm, idx_v, rows_v, gsem, ssem):
    c = lax.axis_index("c")
    s = lax.axis_index("s")
    wid = s * 2 + c
    samp0 = wid * SPW
    pltpu.sync_copy(idx_hbm.at[pl.ds(samp0, SPW)], idx_v)

    nbuf = 4
    nch = SPW * 2

    def gstart(ch):
        return pltpu.async_copy(
            table_hbm.at[idx_v.at[ch // 2, pl.ds((ch % 2) * 32, 32)]],
            rows_v.at[ch % nbuf],
            gsem,
        )

    def sstart(ch):
        return pltpu.async_copy(
            rows_v.at[ch % nbuf], out_hbm.at[pl.ds(samp0 * KP + ch * 32, 32)], ssem
        )

    g = {ch: gstart(ch) for ch in range(nbuf)}
    sc = {}
    for ch in range(nch):
        g[ch].wait()
        sc[ch] = sstart(ch)
        if ch + nbuf < nch:
            sc[ch].wait()
            g[ch + nbuf] = gstart(ch + nbuf)
    for ch in range(nch - nbuf, nch):
        sc[ch].wait()


@functools.cache
def _make_gather():
    return pl.kernel(
        _gather_body,
        out_type=jax.ShapeDtypeStruct((ROWS, D), jnp.float32),
        mesh=plsc.VectorSubcoreMesh(core_axis_name="c", subcore_axis_name="s"),
        scratch_types=[
            pltpu.VMEM((SPW, KP), jnp.int32),
            pltpu.VMEM((4, 32, D), jnp.float32),
            pltpu.SemaphoreType.DMA,
            pltpu.SemaphoreType.DMA,
        ],
    )


_BLKD = 1024
_NBLK = ROWS // _BLKD


def _dense_body(
    rows_ref, l0_wT_ref, l0_b_ref, g_ref, bb_ref, fc_wT_ref, l1_wT_ref, b2_ref,
    out_ref, y_scr, fc_scr, stat_scr,
):
    p = pl.program_id(0)
    i = pl.program_id(1)
    sub = lax.broadcasted_iota(jnp.int32, (_BLKD, 1), 0)
    mask = (sub % KP) < K

    @pl.when(p == 0)
    def _pass0():
        r = rows_ref[...]
        ss = jnp.sum(r * r, axis=1, keepdims=True)
        base = r * lax.rsqrt(ss)
        basem = jnp.where(mask, base, 0.0).astype(jnp.bfloat16)
        y = jnp.dot(basem, l0_wT_ref[...], preferred_element_type=jnp.float32)
        yb = (y + l0_b_ref[...]).astype(jnp.bfloat16)
        y_scr[pl.ds(i * _BLKD, _BLKD), :] = yb
        fc = jnp.dot(basem, fc_wT_ref[...], preferred_element_type=jnp.float32)
        fc_scr[pl.ds(i * _BLKD, _BLKD), :] = fc.astype(jnp.bfloat16)
        ym = jnp.where(mask, yb.astype(jnp.float32), 0.0)

        @pl.when(i == 0)
        def _():
            stat_scr[...] = jnp.zeros_like(stat_scr)

        stat_scr[0:1, :] += jnp.sum(ym, axis=0, keepdims=True)
        stat_scr[1:2, :] += jnp.sum(ym * ym, axis=0, keepdims=True)

    @pl.when(p == 1)
    def _pass1():
        ninv = 1.0 / NSEL
        mean = stat_scr[0:1, :] * ninv
        var = stat_scr[1:2, :] * ninv - mean * mean
        scale = g_ref[...] * lax.rsqrt(var + 1e-5)
        shift = bb_ref[...] - mean * scale
        y = y_scr[pl.ds(i * _BLKD, _BLKD), :].astype(jnp.float32)
        h = (jnp.maximum(y * scale + shift, 0.0)).astype(jnp.bfloat16)
        z = jnp.dot(h, l1_wT_ref[...], preferred_element_type=jnp.float32)
        fc = fc_scr[pl.ds(i * _BLKD, _BLKD), :].astype(jnp.float32)
        f = z + fc + b2_ref[...]
        fm = jnp.where(mask, f, -jnp.inf)
        out_ref[...] = jnp.max(fm.reshape(_BLKD // KP, KP, E), axis=1)


_dense = pl.pallas_call(
    _dense_body,
    grid=(2, _NBLK),
    in_specs=[
        pl.BlockSpec((_BLKD, D), lambda p, i: ((1 - p) * i, 0)),
        pl.BlockSpec((D, H), lambda p, i: (0, 0)),
        pl.BlockSpec((1, H), lambda p, i: (0, 0)),
        pl.BlockSpec((1, H), lambda p, i: (0, 0)),
        pl.BlockSpec((1, H), lambda p, i: (0, 0)),
        pl.BlockSpec((D, E), lambda p, i: (0, 0)),
        pl.BlockSpec((H, E), lambda p, i: (0, 0)),
        pl.BlockSpec((1, E), lambda p, i: (0, 0)),
    ],
    out_specs=pl.BlockSpec((_BLKD // KP, E), lambda p, i: (p * i, 0)),
    out_shape=jax.ShapeDtypeStruct((B, E), jnp.float32),
    scratch_shapes=[
        pltpu.VMEM((ROWS, H), jnp.bfloat16),
        pltpu.VMEM((ROWS, E), jnp.bfloat16),
        pltpu.VMEM((8, H), jnp.float32),
    ],
)


def kernel(all_patch_embeddings, attention_map, fc_w, fc_b, l0_w, l0_b, bn0_g, bn0_b, l1_w, l1_b):
    att = attention_map[:, 0, 1:]
    idx = _select(att)
    table = all_patch_embeddings.reshape(B * T, D)
    rows = _make_gather()(table, idx)
    out = _dense(
        rows,
        l0_w.T.astype(jnp.bfloat16),
        l0_b.reshape(1, H),
        bn0_g.reshape(1, H),
        bn0_b.reshape(1, H),
        fc_w.T.astype(jnp.bfloat16),
        l1_w.T.astype(jnp.bfloat16),
        (l1_b + fc_b).reshape(1, E),
    )
    return out
